# baseline (speedup 1.0000x reference)
import functools

import jax
import jax.numpy as jnp
from jax.experimental import pallas as pl
from jax.experimental.pallas import tpu as pltpu

N = 10000
BQ = 1024
TQ = (N + BQ - 1) // BQ
REM = N - (TQ - 1) * BQ


def _gcn_body(adj_ref, x_ref, w1_ref, b1_ref, w2_ref, b2_ref, out_ref,
              s1_ref, s2_ref, hacc_ref, oacc_ref):
    p = pl.program_id(0)
    i = pl.program_id(1)
    kb = pl.program_id(2)

    @pl.when((p == 0) & (i == 0) & (kb == 0))
    def _compute_s1():
        s1_ref[:N, :] = jnp.dot(x_ref[...], w1_ref[...],
                                preferred_element_type=jnp.float32)

    @pl.when(p == 0)
    def _sweep1():
        a = adj_ref[...]

        @pl.when(kb == 0)
        def _():
            hacc_ref[...] = jnp.dot(a, s1_ref[pl.ds(0, BQ), :],
                                    preferred_element_type=jnp.float32)

        @pl.when((kb > 0) & (kb < TQ - 1))
        def _():
            hacc_ref[...] = hacc_ref[...] + jnp.dot(
                a, s1_ref[pl.ds(kb * BQ, BQ), :],
                preferred_element_type=jnp.float32)

        @pl.when((kb == 0) & (i > 0))
        def _():
            oacc_ref[pl.ds(i * BQ, BQ), :] = jnp.dot(
                a, s2_ref[pl.ds(0, BQ), :],
                preferred_element_type=jnp.float32)

        @pl.when((kb > 0) & (kb < i))
        def _():
            oacc_ref[pl.ds(i * BQ, BQ), :] = (
                oacc_ref[pl.ds(i * BQ, BQ), :]
                + jnp.dot(a, s2_ref[pl.ds(kb * BQ, BQ), :],
                          preferred_element_type=jnp.float32))

        @pl.when(kb == TQ - 1)
        def _():
            hacc = hacc_ref[...] + jnp.dot(
                a[:, :REM], s1_ref[pl.ds((TQ - 1) * BQ, REM), :],
                preferred_element_type=jnp.float32)
            h = jnp.maximum(hacc + b1_ref[...], 0.0)
            s2_ref[pl.ds(i * BQ, BQ), :] = jnp.dot(
                h, w2_ref[...], preferred_element_type=jnp.float32)

    @pl.when((p == 1) & (kb >= i))
    def _sweep2():
        a = adj_ref[...]

        @pl.when((kb == 0) & (i == 0))
        def _():
            oacc_ref[pl.ds(0, BQ), :] = jnp.dot(
                a, s2_ref[pl.ds(0, BQ), :],
                preferred_element_type=jnp.float32)

        @pl.when((kb > 0) & (kb < TQ - 1))
        def _():
            oacc_ref[pl.ds(i * BQ, BQ), :] = (
                oacc_ref[pl.ds(i * BQ, BQ), :]
                + jnp.dot(a, s2_ref[pl.ds(kb * BQ, BQ), :],
                          preferred_element_type=jnp.float32))

        @pl.when(kb == TQ - 1)
        def _():
            oacc = (oacc_ref[pl.ds(i * BQ, BQ), :]
                    + jnp.dot(a[:, :REM],
                              s2_ref[pl.ds((TQ - 1) * BQ, REM), :],
                              preferred_element_type=jnp.float32))
            oacc_ref[pl.ds(i * BQ, BQ), :] = oacc
            out_ref[...] = oacc + b2_ref[...]


@functools.partial(jax.jit, static_argnames=("interpret",))
def _gcn(x, adj, W1, b1, W2, b2, interpret=False):
    nfeat = x.shape[1]
    nhid = W1.shape[1]
    nclass = W2.shape[1]

    def adj_map(p, i, kb):
        return (i, jnp.where(p == 0, kb, jnp.maximum(kb, i)))

    return pl.pallas_call(
        _gcn_body,
        grid=(2, TQ, TQ),
        in_specs=[
            pl.BlockSpec((BQ, BQ), adj_map),
            pl.BlockSpec((N, nfeat), lambda p, i, kb: (0, 0)),
            pl.BlockSpec((nfeat, nhid), lambda p, i, kb: (0, 0)),
            pl.BlockSpec((1, nhid), lambda p, i, kb: (0, 0)),
            pl.BlockSpec((nhid, nclass), lambda p, i, kb: (0, 0)),
            pl.BlockSpec((1, nclass), lambda p, i, kb: (0, 0)),
        ],
        out_specs=pl.BlockSpec((BQ, nclass), lambda p, i, kb: (i, 0)),
        out_shape=jax.ShapeDtypeStruct((N, nclass), jnp.float32),
        scratch_shapes=[
            pltpu.VMEM((TQ * BQ, nhid), jnp.float32),
            pltpu.VMEM((TQ * BQ, nclass), jnp.float32),
            pltpu.VMEM((BQ, nhid), jnp.float32),
            pltpu.VMEM((TQ * BQ, nclass), jnp.float32),
        ],
        interpret=interpret,
    )(adj, x, W1, b1.reshape(1, -1), W2, b2.reshape(1, -1))


def kernel(x, adj, W1, b1, W2, b2):
    return _gcn(x, adj, W1, b1, W2, b2)


# speedup vs baseline: 1.1349x; 1.1349x over previous
import functools

import jax
import jax.numpy as jnp
from jax.experimental import pallas as pl
from jax.experimental.pallas import tpu as pltpu

N = 10000
BI = 200
BX = 2000


def _s1_body(x_ref, w1_ref, s1_ref):
    s1_ref[...] = jnp.dot(x_ref[...], w1_ref[...],
                          preferred_element_type=jnp.float32)


def _gcn_body(adj_ref, s1_ref, b1_ref, w2_ref, b2_ref, out_ref, s2_ref):
    p = pl.program_id(0)
    i = pl.program_id(1)

    @pl.when(p == 0)
    def _layer1():
        h = jnp.dot(adj_ref[...], s1_ref[...],
                    preferred_element_type=jnp.float32) + b1_ref[...]
        h = jnp.maximum(h, 0.0)
        s2_ref[pl.ds(i * BI, BI), :] = jnp.dot(
            h, w2_ref[...], preferred_element_type=jnp.float32)

    @pl.when(p == 1)
    def _layer2():
        out_ref[...] = jnp.dot(adj_ref[...], s2_ref[...],
                               preferred_element_type=jnp.float32) + b2_ref[...]


@functools.partial(jax.jit, static_argnames=("interpret",))
def _gcn(x, adj, W1, b1, W2, b2, interpret=False):
    nfeat = x.shape[1]
    nhid = W1.shape[1]
    nclass = W2.shape[1]

    s1 = pl.pallas_call(
        _s1_body,
        grid=(N // BX,),
        in_specs=[
            pl.BlockSpec((BX, nfeat), lambda i: (i, 0)),
            pl.BlockSpec((nfeat, nhid), lambda i: (0, 0)),
        ],
        out_specs=pl.BlockSpec((BX, nhid), lambda i: (i, 0)),
        out_shape=jax.ShapeDtypeStruct((N, nhid), jnp.float32),
        interpret=interpret,
    )(x, W1)

    return pl.pallas_call(
        _gcn_body,
        grid=(2, N // BI),
        in_specs=[
            pl.BlockSpec((BI, N), lambda p, i: (i, 0)),
            pl.BlockSpec((N, nhid), lambda p, i: (0, 0)),
            pl.BlockSpec((1, nhid), lambda p, i: (0, 0)),
            pl.BlockSpec((nhid, nclass), lambda p, i: (0, 0)),
            pl.BlockSpec((1, nclass), lambda p, i: (0, 0)),
        ],
        out_specs=pl.BlockSpec(
            (BI, nclass), lambda p, i: (jnp.where(p == 1, i, 0), 0)),
        out_shape=jax.ShapeDtypeStruct((N, nclass), jnp.float32),
        scratch_shapes=[
            pltpu.VMEM((N, nclass), jnp.float32),
        ],
        interpret=interpret,
    )(adj, s1, b1.reshape(1, -1), W2, b2.reshape(1, -1))


def kernel(x, adj, W1, b1, W2, b2):
    return _gcn(x, adj, W1, b1, W2, b2)


# speedup vs baseline: 1.1582x; 1.0206x over previous
import functools

import jax
import jax.numpy as jnp
from jax.experimental import pallas as pl
from jax.experimental.pallas import tpu as pltpu

N = 10000
BI = 400
BX = 2000


def _s1_body(x_ref, w1_ref, s1_ref):
    s1_ref[...] = jnp.dot(x_ref[...], w1_ref[...],
                          preferred_element_type=jnp.float32)


def _gcn_body(adj_ref, s1_ref, b1_ref, w2_ref, b2_ref, out_ref, s2_ref):
    p = pl.program_id(0)
    i = pl.program_id(1)

    @pl.when(p == 0)
    def _layer1():
        h = jnp.dot(adj_ref[...], s1_ref[...],
                    preferred_element_type=jnp.float32) + b1_ref[...]
        h = jnp.maximum(h, 0.0)
        s2_ref[pl.ds(i * BI, BI), :] = jnp.dot(
            h, w2_ref[...], preferred_element_type=jnp.float32)

    @pl.when(p == 1)
    def _layer2():
        out_ref[...] = jnp.dot(adj_ref[...], s2_ref[...],
                               preferred_element_type=jnp.float32) + b2_ref[...]


@functools.partial(jax.jit, static_argnames=("interpret",))
def _gcn(x, adj, W1, b1, W2, b2, interpret=False):
    nfeat = x.shape[1]
    nhid = W1.shape[1]
    nclass = W2.shape[1]

    s1 = pl.pallas_call(
        _s1_body,
        grid=(N // BX,),
        in_specs=[
            pl.BlockSpec((BX, nfeat), lambda i: (i, 0)),
            pl.BlockSpec((nfeat, nhid), lambda i: (0, 0)),
        ],
        out_specs=pl.BlockSpec((BX, nhid), lambda i: (i, 0)),
        out_shape=jax.ShapeDtypeStruct((N, nhid), jnp.float32),
        interpret=interpret,
    )(x, W1)

    return pl.pallas_call(
        _gcn_body,
        grid=(2, N // BI),
        in_specs=[
            pl.BlockSpec((BI, N), lambda p, i: (i, 0)),
            pl.BlockSpec((N, nhid), lambda p, i: (0, 0)),
            pl.BlockSpec((1, nhid), lambda p, i: (0, 0)),
            pl.BlockSpec((nhid, nclass), lambda p, i: (0, 0)),
            pl.BlockSpec((1, nclass), lambda p, i: (0, 0)),
        ],
        out_specs=pl.BlockSpec(
            (BI, nclass), lambda p, i: (jnp.where(p == 1, i, 0), 0)),
        out_shape=jax.ShapeDtypeStruct((N, nclass), jnp.float32),
        scratch_shapes=[
            pltpu.VMEM((N, nclass), jnp.float32),
        ],
        interpret=interpret,
    )(adj, s1, b1.reshape(1, -1), W2, b2.reshape(1, -1))


def kernel(x, adj, W1, b1, W2, b2):
    return _gcn(x, adj, W1, b1, W2, b2)
